# Initial kernel scaffold; baseline (speedup 1.0000x reference)
import jax
import jax.numpy as jnp
from jax.experimental import pallas as pl


def kernel(feats, edge_index, W1, b1, Wp, bp, W2, b2):
    raise NotImplementedError("write your pallas kernel here")


# speedup vs baseline: 5.7742x; 5.7742x over previous
import functools

import jax
import jax.numpy as jnp
from jax import lax
from jax.experimental import pallas as pl
from jax.experimental.pallas import tpu as pltpu
from jax.experimental.pallas import tpu_sc as plsc

N = 10000
E = 320000
F = 128
P = 4

NC = 2
NS = 16
NW = NC * NS
EPW = E // NW
CH = 128
NFULL = EPW // CH
TAIL = EPW - NFULL * CH
RPT = 624
RTAIL = N - NS * RPT
BLK = 2000


def _mesh():
    return plsc.VectorSubcoreMesh(core_axis_name="c", subcore_axis_name="s")


_SC_PARAMS = pltpu.CompilerParams(needs_layout_passes=False)


def _deg_body(src_hbm, dst_hbm, dout_hbm, din_hbm, esrc_v, edst_v, dout_v, din_v):
    c = lax.axis_index("c")
    s = lax.axis_index("s")
    wid = s * NC + c
    base = wid * EPW
    zero = jnp.zeros((16,), jnp.int32)
    one = jnp.ones((16,), jnp.int32)

    def zi(i, carry):
        dout_v[pl.ds(i * 16, 16)] = zero
        din_v[pl.ds(i * 16, 16)] = zero
        return carry

    lax.fori_loop(0, N // 16, zi, 0)

    pltpu.sync_copy(src_hbm.at[pl.ds(base, EPW)], esrc_v)
    pltpu.sync_copy(dst_hbm.at[pl.ds(base, EPW)], edst_v)

    def acc(i, carry):
        si = esrc_v[pl.ds(i * 16, 16)]
        plsc.addupdate_scatter(dout_v, [si], one)
        di = edst_v[pl.ds(i * 16, 16)]
        plsc.addupdate_scatter(din_v, [di], one)
        return carry

    lax.fori_loop(0, EPW // 16, acc, 0)

    pltpu.sync_copy(dout_v, dout_hbm.at[pl.ds(wid * N, N)])
    pltpu.sync_copy(din_v, din_hbm.at[pl.ds(wid * N, N)])


_deg_kernel = functools.partial(
    pl.kernel,
    out_type=[
        jax.ShapeDtypeStruct((NW * N,), jnp.int32),
        jax.ShapeDtypeStruct((NW * N,), jnp.int32),
    ],
    mesh=_mesh(),
    compiler_params=_SC_PARAMS,
    scratch_types=[
        pltpu.VMEM((EPW,), jnp.int32),
        pltpu.VMEM((EPW,), jnp.int32),
        pltpu.VMEM((N,), jnp.int32),
        pltpu.VMEM((N,), jnp.int32),
    ],
)(_deg_body)


def _zero_zbuf(zbuf_v):
    zf = jnp.zeros((16,), jnp.float32)
    for r in range(16):
        for k in range(8):
            zbuf_v[r, pl.ds(k * 16, 16)] = zf


def _zero_acc_slice(zbuf_v, acc_sh, r0, s):
    def za(j, carry):
        pltpu.sync_copy(zbuf_v, acc_sh.at[pl.ds(r0 + j * 16, 16)])
        return carry

    lax.fori_loop(0, RPT // 16, za, 0)

    @pl.when(s == 0)
    def _():
        pltpu.sync_copy(zbuf_v, acc_sh.at[pl.ds(NS * RPT, RTAIL)])


def _pass1_body(x_hbm, src_hbm, dst_hbm, sout_hbm, part_hbm, tpart_hbm,
                sidx_v, didx_v, rows_v, sidx2_v, didx2_v, rows2_v,
                zbuf_v, sout_v, t_v, acc_sh, sem):
    c = lax.axis_index("c")
    s = lax.axis_index("s")
    wid = s * NC + c
    base = wid * EPW
    r0 = s * RPT
    zf = jnp.zeros((16,), jnp.float32)

    _zero_zbuf(zbuf_v)

    def zt(i, carry):
        t_v[pl.ds(i * 16, 16)] = zf
        return carry

    lax.fori_loop(0, N // 16, zt, 0)

    _zero_acc_slice(zbuf_v, acc_sh, r0, s)
    pltpu.sync_copy(sout_hbm, sout_v)
    plsc.subcore_barrier()

    def chunk(j, carry):
        off = base + j * CH
        pltpu.sync_copy(src_hbm.at[pl.ds(off, CH)], sidx_v)
        pltpu.sync_copy(dst_hbm.at[pl.ds(off, CH)], didx_v)
        pltpu.async_copy(x_hbm.at[sidx_v], rows_v, sem).wait()
        pltpu.sync_copy(rows_v, acc_sh.at[didx_v], add=True)
        for k in range(CH // 16):
            si = sidx_v[pl.ds(k * 16, 16)]
            vals = plsc.load_gather(sout_v, [si])
            di = didx_v[pl.ds(k * 16, 16)]
            plsc.addupdate_scatter(t_v, [di], vals)
        return carry

    lax.fori_loop(0, NFULL, chunk, 0)

    off = base + NFULL * CH
    pltpu.sync_copy(src_hbm.at[pl.ds(off, TAIL)], sidx2_v)
    pltpu.sync_copy(dst_hbm.at[pl.ds(off, TAIL)], didx2_v)
    pltpu.async_copy(x_hbm.at[sidx2_v], rows2_v, sem).wait()
    pltpu.sync_copy(rows2_v, acc_sh.at[didx2_v], add=True)
    si = sidx2_v[...]
    vals = plsc.load_gather(sout_v, [si])
    plsc.addupdate_scatter(t_v, [didx2_v[...]], vals)

    plsc.subcore_barrier()
    pltpu.sync_copy(acc_sh.at[pl.ds(r0, RPT)], part_hbm.at[c, pl.ds(r0, RPT)])

    @pl.when(s == 0)
    def _():
        pltpu.sync_copy(
            acc_sh.at[pl.ds(NS * RPT, RTAIL)], part_hbm.at[c, pl.ds(NS * RPT, RTAIL)]
        )

    pltpu.sync_copy(t_v, tpart_hbm.at[pl.ds(wid * N, N)])


_pass1_kernel = functools.partial(
    pl.kernel,
    out_type=[
        jax.ShapeDtypeStruct((NC, N, F), jnp.float32),
        jax.ShapeDtypeStruct((NW * N,), jnp.float32),
    ],
    mesh=_mesh(),
    compiler_params=_SC_PARAMS,
    scratch_types=[
        pltpu.VMEM((CH,), jnp.int32),
        pltpu.VMEM((CH,), jnp.int32),
        pltpu.VMEM((CH, F), jnp.float32),
        pltpu.VMEM((TAIL,), jnp.int32),
        pltpu.VMEM((TAIL,), jnp.int32),
        pltpu.VMEM((TAIL, F), jnp.float32),
        pltpu.VMEM((16, F), jnp.float32),
        pltpu.VMEM((N,), jnp.float32),
        pltpu.VMEM((N,), jnp.float32),
        pltpu.VMEM_SHARED((N, F), jnp.float32),
        pltpu.SemaphoreType.DMA,
    ],
)(_pass1_body)


def _pass2_body(x_hbm, src_hbm, dst_hbm, part_hbm,
                sidx_v, didx_v, rows_v, sidx2_v, didx2_v, rows2_v,
                zbuf_v, acc_sh, sem):
    c = lax.axis_index("c")
    s = lax.axis_index("s")
    wid = s * NC + c
    base = wid * EPW
    r0 = s * RPT

    _zero_zbuf(zbuf_v)
    _zero_acc_slice(zbuf_v, acc_sh, r0, s)
    plsc.subcore_barrier()

    def chunk(j, carry):
        off = base + j * CH
        pltpu.sync_copy(src_hbm.at[pl.ds(off, CH)], sidx_v)
        pltpu.sync_copy(dst_hbm.at[pl.ds(off, CH)], didx_v)
        pltpu.async_copy(x_hbm.at[sidx_v], rows_v, sem).wait()
        pltpu.sync_copy(rows_v, acc_sh.at[didx_v], add=True)
        return carry

    lax.fori_loop(0, NFULL, chunk, 0)

    off = base + NFULL * CH
    pltpu.sync_copy(src_hbm.at[pl.ds(off, TAIL)], sidx2_v)
    pltpu.sync_copy(dst_hbm.at[pl.ds(off, TAIL)], didx2_v)
    pltpu.async_copy(x_hbm.at[sidx2_v], rows2_v, sem).wait()
    pltpu.sync_copy(rows2_v, acc_sh.at[didx2_v], add=True)

    plsc.subcore_barrier()
    pltpu.sync_copy(acc_sh.at[pl.ds(r0, RPT)], part_hbm.at[c, pl.ds(r0, RPT)])

    @pl.when(s == 0)
    def _():
        pltpu.sync_copy(
            acc_sh.at[pl.ds(NS * RPT, RTAIL)], part_hbm.at[c, pl.ds(NS * RPT, RTAIL)]
        )


_pass2_kernel = functools.partial(
    pl.kernel,
    out_type=jax.ShapeDtypeStruct((NC, N, F), jnp.float32),
    mesh=_mesh(),
    compiler_params=_SC_PARAMS,
    scratch_types=[
        pltpu.VMEM((CH,), jnp.int32),
        pltpu.VMEM((CH,), jnp.int32),
        pltpu.VMEM((CH, F), jnp.float32),
        pltpu.VMEM((TAIL,), jnp.int32),
        pltpu.VMEM((TAIL,), jnp.int32),
        pltpu.VMEM((TAIL, F), jnp.float32),
        pltpu.VMEM((16, F), jnp.float32),
        pltpu.VMEM_SHARED((N, F), jnp.float32),
        pltpu.SemaphoreType.DMA,
    ],
)(_pass2_body)


def _prescale_body(dot_ref, dit_ref, x_ref, xs_ref, so_ref, si_ref):
    do = jnp.sum(dot_ref[...].astype(jnp.float32), axis=1, keepdims=True)
    so = lax.rsqrt(jnp.maximum(do, 1.0))
    di = jnp.sum(dit_ref[...].astype(jnp.float32), axis=1, keepdims=True)
    si = lax.rsqrt(jnp.maximum(di, 1.0))
    xs_ref[...] = x_ref[...] * so
    so_ref[...] = so
    si_ref[...] = si


def _prescale_call(dout_t, din_t, feats):
    grid = (N // BLK,)
    return pl.pallas_call(
        _prescale_body,
        grid=grid,
        in_specs=[
            pl.BlockSpec((BLK, NW), lambda i: (i, 0)),
            pl.BlockSpec((BLK, NW), lambda i: (i, 0)),
            pl.BlockSpec((BLK, F), lambda i: (i, 0)),
        ],
        out_specs=[
            pl.BlockSpec((BLK, F), lambda i: (i, 0)),
            pl.BlockSpec((BLK, 1), lambda i: (i, 0)),
            pl.BlockSpec((BLK, 1), lambda i: (i, 0)),
        ],
        out_shape=[
            jax.ShapeDtypeStruct((N, F), jnp.float32),
            jax.ShapeDtypeStruct((N, 1), jnp.float32),
            jax.ShapeDtypeStruct((N, 1), jnp.float32),
        ],
    )(dout_t, din_t, feats)


def _mid_body(m1a_ref, m1b_ref, tt_ref, si_ref, so_ref, w1_ref, b1_ref,
              wp_ref, bp_ref, w2b_ref, b2_ref, h1s_ref, r_ref):
    m1 = m1a_ref[...] + m1b_ref[...]
    sic = si_ref[...]
    a = m1 * sic
    h1 = jnp.dot(a, w1_ref[...], preferred_element_type=jnp.float32) + b1_ref[...]
    h1 = jnp.maximum(h1, 0.0)
    h1s_ref[...] = h1 * so_ref[...]
    tcol = jnp.sum(tt_ref[...], axis=1, keepdims=True)
    z = jnp.dot(a, wp_ref[...], preferred_element_type=jnp.float32)
    z = z + (tcol * sic) * bp_ref[...]
    r_ref[...] = (
        jnp.dot(z, w2b_ref[...], preferred_element_type=jnp.float32) + b2_ref[...]
    )


def _mid_call(m1a, m1b, t_t, si, so, W1, b1r, Wp, bpr, W2b, b2r):
    grid = (N // BLK,)
    full = lambda i: (0, 0)
    return pl.pallas_call(
        _mid_body,
        grid=grid,
        in_specs=[
            pl.BlockSpec((BLK, F), lambda i: (i, 0)),
            pl.BlockSpec((BLK, F), lambda i: (i, 0)),
            pl.BlockSpec((BLK, NW), lambda i: (i, 0)),
            pl.BlockSpec((BLK, 1), lambda i: (i, 0)),
            pl.BlockSpec((BLK, 1), lambda i: (i, 0)),
            pl.BlockSpec((F, F), full),
            pl.BlockSpec((1, F), full),
            pl.BlockSpec((F, P), full),
            pl.BlockSpec((1, P), full),
            pl.BlockSpec((P, F), full),
            pl.BlockSpec((1, F), full),
        ],
        out_specs=[
            pl.BlockSpec((BLK, F), lambda i: (i, 0)),
            pl.BlockSpec((BLK, F), lambda i: (i, 0)),
        ],
        out_shape=[
            jax.ShapeDtypeStruct((N, F), jnp.float32),
            jax.ShapeDtypeStruct((N, F), jnp.float32),
        ],
    )(m1a, m1b, t_t, si, so, W1, b1r, Wp, bpr, W2b, b2r)


def _fin_body(m2a_ref, m2b_ref, r_ref, si_ref, w2a_ref, out_ref):
    m2 = (m2a_ref[...] + m2b_ref[...]) * si_ref[...]
    out_ref[...] = (
        jnp.dot(m2, w2a_ref[...], preferred_element_type=jnp.float32) + r_ref[...]
    )


def _fin_call(m2a, m2b, r, si, W2a):
    grid = (N // BLK,)
    full = lambda i: (0, 0)
    return pl.pallas_call(
        _fin_body,
        grid=grid,
        in_specs=[
            pl.BlockSpec((BLK, F), lambda i: (i, 0)),
            pl.BlockSpec((BLK, F), lambda i: (i, 0)),
            pl.BlockSpec((BLK, F), lambda i: (i, 0)),
            pl.BlockSpec((BLK, 1), lambda i: (i, 0)),
            pl.BlockSpec((F, F), full),
        ],
        out_specs=pl.BlockSpec((BLK, F), lambda i: (i, 0)),
        out_shape=jax.ShapeDtypeStruct((N, F), jnp.float32),
    )(m2a, m2b, r, si, W2a)


def kernel(feats, edge_index, W1, b1, Wp, bp, W2, b2):
    src = edge_index[0]
    dst = edge_index[1]

    dout_p, din_p = _deg_kernel(src, dst)
    xs, so, si = _prescale_call(dout_p.reshape(NW, N).T, din_p.reshape(NW, N).T, feats)

    m1_p, t_p = _pass1_kernel(xs, src, dst, so.reshape(N))
    h1s, r = _mid_call(
        m1_p[0], m1_p[1], t_p.reshape(NW, N).T, si, so,
        W1, b1.reshape(1, F), Wp, bp.reshape(1, P), W2[F:], b2.reshape(1, F),
    )

    m2_p = _pass2_kernel(h1s, src, dst)
    out = _fin_call(m2_p[0], m2_p[1], r, si, W2[:F])
    return out


# speedup vs baseline: 8.0149x; 1.3880x over previous
import functools

import jax
import jax.numpy as jnp
from jax import lax
from jax.experimental import pallas as pl
from jax.experimental.pallas import tpu as pltpu
from jax.experimental.pallas import tpu_sc as plsc

N = 10000
E = 320000
F = 128
P = 4

NC = 2
NS = 16
NW = NC * NS
EPW = E // NW
CH1 = 96
CH2 = 128
TAIL = 16
RPT = 624
RTAIL = N - NS * RPT
BLK = 2000


def _mesh():
    return plsc.VectorSubcoreMesh(core_axis_name="c", subcore_axis_name="s")


_SC_PARAMS = pltpu.CompilerParams(needs_layout_passes=False)


def _deg_body(src_hbm, dst_hbm, dout_hbm, din_hbm, esrc_v, edst_v, dout_v, din_v):
    c = lax.axis_index("c")
    s = lax.axis_index("s")
    wid = s * NC + c
    base = wid * EPW
    zero = jnp.zeros((16,), jnp.int32)
    one = jnp.ones((16,), jnp.int32)

    def zi(i, carry):
        dout_v[pl.ds(i * 16, 16)] = zero
        din_v[pl.ds(i * 16, 16)] = zero
        return carry

    lax.fori_loop(0, N // 16, zi, 0)

    pltpu.sync_copy(src_hbm.at[pl.ds(base, EPW)], esrc_v)
    pltpu.sync_copy(dst_hbm.at[pl.ds(base, EPW)], edst_v)

    def acc(i, carry):
        si = esrc_v[pl.ds(i * 16, 16)]
        plsc.addupdate_scatter(dout_v, [si], one)
        di = edst_v[pl.ds(i * 16, 16)]
        plsc.addupdate_scatter(din_v, [di], one)
        return carry

    lax.fori_loop(0, EPW // 16, acc, 0)

    pltpu.sync_copy(dout_v, dout_hbm.at[pl.ds(wid * N, N)])
    pltpu.sync_copy(din_v, din_hbm.at[pl.ds(wid * N, N)])


_deg_kernel = functools.partial(
    pl.kernel,
    out_type=[
        jax.ShapeDtypeStruct((NW * N,), jnp.int32),
        jax.ShapeDtypeStruct((NW * N,), jnp.int32),
    ],
    mesh=_mesh(),
    compiler_params=_SC_PARAMS,
    scratch_types=[
        pltpu.VMEM((EPW,), jnp.int32),
        pltpu.VMEM((EPW,), jnp.int32),
        pltpu.VMEM((N,), jnp.int32),
        pltpu.VMEM((N,), jnp.int32),
    ],
)(_deg_body)


def _zero_zbuf(zbuf_v):
    zf = jnp.zeros((16,), jnp.float32)
    for r in range(8):
        for k in range(8):
            zbuf_v[r, pl.ds(k * 16, 16)] = zf


def _zero_acc_slice(zbuf_v, acc_sh, r0, s):
    def za(j, carry):
        pltpu.sync_copy(zbuf_v, acc_sh.at[pl.ds(r0 + j * 8, 8)])
        return carry

    lax.fori_loop(0, RPT // 8, za, 0)

    @pl.when(s == 0)
    def _():
        pltpu.sync_copy(zbuf_v, acc_sh.at[pl.ds(NS * RPT, 8)])
        pltpu.sync_copy(zbuf_v, acc_sh.at[pl.ds(NS * RPT + 8, 8)])


def _t_update(sidx_v, didx_v, sout_v, t_v, ngroups):
    for k in range(ngroups):
        si = sidx_v[pl.ds(k * 16, 16)]
        vals = plsc.load_gather(sout_v, [si])
        di = didx_v[pl.ds(k * 16, 16)]
        plsc.addupdate_scatter(t_v, [di], vals)


def _edge_pass_body(has_t, ch, refs):
    nfull = (EPW - TAIL) // ch
    npair = nfull // 2
    if has_t:
        (x_hbm, src_hbm, dst_hbm, sout_hbm, part_hbm, tpart_hbm,
         sA, dA, rA, sB, dB, rB, s2, d2, zbuf_v, sout_v, t_v, acc_sh,
         gsA, gsB, ssc) = refs
    else:
        (x_hbm, src_hbm, dst_hbm, part_hbm,
         sA, dA, rA, sB, dB, rB, s2, d2, zbuf_v, acc_sh,
         gsA, gsB, ssc) = refs
        sout_v = t_v = None

    c = lax.axis_index("c")
    s = lax.axis_index("s")
    wid = s * NC + c
    base = wid * EPW
    r0 = s * RPT
    zf = jnp.zeros((16,), jnp.float32)

    _zero_zbuf(zbuf_v)

    if has_t:
        def zt(i, carry):
            t_v[pl.ds(i * 16, 16)] = zf
            return carry

        lax.fori_loop(0, N // 16, zt, 0)
        pltpu.sync_copy(sout_hbm, sout_v)

    _zero_acc_slice(zbuf_v, acc_sh, r0, s)
    plsc.subcore_barrier()

    def loadidx(j, si, di):
        off = base + j * ch
        pltpu.sync_copy(src_hbm.at[pl.ds(off, ch)], si)
        pltpu.sync_copy(dst_hbm.at[pl.ds(off, ch)], di)

    loadidx(0, sA, dA)
    pltpu.async_copy(x_hbm.at[sA], rA, gsA)
    loadidx(1, sB, dB)
    pltpu.async_copy(x_hbm.at[sB], rB, gsB)

    def half(j_next, si, di, rr, gs):
        pltpu.make_async_copy(x_hbm.at[si], rr, gs).wait()
        desc = pltpu.async_copy(rr, acc_sh.at[di], ssc, add=True)
        if has_t:
            _t_update(si, di, sout_v, t_v, ch // 16)
        desc.wait()

        @pl.when(j_next < nfull)
        def _():
            loadidx(j_next, si, di)
            pltpu.async_copy(x_hbm.at[si], rr, gs)

    def pair(p, carry):
        half(2 * p + 2, sA, dA, rA, gsA)
        half(2 * p + 3, sB, dB, rB, gsB)
        return carry

    lax.fori_loop(0, npair, pair, 0)

    off = base + nfull * ch
    pltpu.sync_copy(src_hbm.at[pl.ds(off, TAIL)], s2)
    pltpu.sync_copy(dst_hbm.at[pl.ds(off, TAIL)], d2)
    pltpu.async_copy(x_hbm.at[s2], rA.at[pl.ds(0, TAIL)], gsA).wait()
    pltpu.sync_copy(rA.at[pl.ds(0, TAIL)], acc_sh.at[d2], add=True)
    if has_t:
        _t_update(s2, d2, sout_v, t_v, TAIL // 16)

    plsc.subcore_barrier()
    pltpu.sync_copy(acc_sh.at[pl.ds(r0, RPT)], part_hbm.at[c, pl.ds(r0, RPT)])

    @pl.when(s == 0)
    def _():
        pltpu.sync_copy(
            acc_sh.at[pl.ds(NS * RPT, RTAIL)], part_hbm.at[c, pl.ds(NS * RPT, RTAIL)]
        )

    if has_t:
        pltpu.sync_copy(t_v, tpart_hbm.at[pl.ds(wid * N, N)])


def _pass1_body(*refs):
    _edge_pass_body(True, CH1, refs)


def _pass2_body(*refs):
    _edge_pass_body(False, CH2, refs)


def _pipe_scratch(ch):
    return [
        pltpu.VMEM((ch,), jnp.int32),
        pltpu.VMEM((ch,), jnp.int32),
        pltpu.VMEM((ch, F), jnp.float32),
        pltpu.VMEM((ch,), jnp.int32),
        pltpu.VMEM((ch,), jnp.int32),
        pltpu.VMEM((ch, F), jnp.float32),
        pltpu.VMEM((TAIL,), jnp.int32),
        pltpu.VMEM((TAIL,), jnp.int32),
        pltpu.VMEM((8, F), jnp.float32),
    ]

_pass1_kernel = functools.partial(
    pl.kernel,
    out_type=[
        jax.ShapeDtypeStruct((NC, N, F), jnp.float32),
        jax.ShapeDtypeStruct((NW * N,), jnp.float32),
    ],
    mesh=_mesh(),
    compiler_params=_SC_PARAMS,
    scratch_types=_pipe_scratch(CH1) + [
        pltpu.VMEM((N,), jnp.float32),
        pltpu.VMEM((N,), jnp.float32),
        pltpu.VMEM_SHARED((N, F), jnp.float32),
        pltpu.SemaphoreType.DMA,
        pltpu.SemaphoreType.DMA,
        pltpu.SemaphoreType.DMA,
    ],
)(_pass1_body)

_pass2_kernel = functools.partial(
    pl.kernel,
    out_type=jax.ShapeDtypeStruct((NC, N, F), jnp.float32),
    mesh=_mesh(),
    compiler_params=_SC_PARAMS,
    scratch_types=_pipe_scratch(CH2) + [
        pltpu.VMEM_SHARED((N, F), jnp.float32),
        pltpu.SemaphoreType.DMA,
        pltpu.SemaphoreType.DMA,
        pltpu.SemaphoreType.DMA,
    ],
)(_pass2_body)


def _prescale_body(dot_ref, dit_ref, x_ref, xs_ref, so_ref, si_ref):
    do = jnp.sum(dot_ref[...].astype(jnp.float32), axis=1, keepdims=True)
    so = lax.rsqrt(jnp.maximum(do, 1.0))
    di = jnp.sum(dit_ref[...].astype(jnp.float32), axis=1, keepdims=True)
    si = lax.rsqrt(jnp.maximum(di, 1.0))
    xs_ref[...] = x_ref[...] * so
    so_ref[...] = so
    si_ref[...] = si


def _prescale_call(dout_t, din_t, feats):
    grid = (N // BLK,)
    return pl.pallas_call(
        _prescale_body,
        grid=grid,
        in_specs=[
            pl.BlockSpec((BLK, NW), lambda i: (i, 0)),
            pl.BlockSpec((BLK, NW), lambda i: (i, 0)),
            pl.BlockSpec((BLK, F), lambda i: (i, 0)),
        ],
        out_specs=[
            pl.BlockSpec((BLK, F), lambda i: (i, 0)),
            pl.BlockSpec((BLK, 1), lambda i: (i, 0)),
            pl.BlockSpec((BLK, 1), lambda i: (i, 0)),
        ],
        out_shape=[
            jax.ShapeDtypeStruct((N, F), jnp.float32),
            jax.ShapeDtypeStruct((N, 1), jnp.float32),
            jax.ShapeDtypeStruct((N, 1), jnp.float32),
        ],
    )(dout_t, din_t, feats)


def _mid_body(m1a_ref, m1b_ref, tt_ref, si_ref, so_ref, w1_ref, b1_ref,
              wp_ref, bp_ref, w2b_ref, b2_ref, h1s_ref, r_ref):
    m1 = m1a_ref[...] + m1b_ref[...]
    sic = si_ref[...]
    a = m1 * sic
    h1 = jnp.dot(a, w1_ref[...], preferred_element_type=jnp.float32) + b1_ref[...]
    h1 = jnp.maximum(h1, 0.0)
    h1s_ref[...] = h1 * so_ref[...]
    tcol = jnp.sum(tt_ref[...], axis=1, keepdims=True)
    z = jnp.dot(a, wp_ref[...], preferred_element_type=jnp.float32)
    z = z + (tcol * sic) * bp_ref[...]
    r_ref[...] = (
        jnp.dot(z, w2b_ref[...], preferred_element_type=jnp.float32) + b2_ref[...]
    )


def _mid_call(m1a, m1b, t_t, si, so, W1, b1r, Wp, bpr, W2b, b2r):
    grid = (N // BLK,)
    full = lambda i: (0, 0)
    return pl.pallas_call(
        _mid_body,
        grid=grid,
        in_specs=[
            pl.BlockSpec((BLK, F), lambda i: (i, 0)),
            pl.BlockSpec((BLK, F), lambda i: (i, 0)),
            pl.BlockSpec((BLK, NW), lambda i: (i, 0)),
            pl.BlockSpec((BLK, 1), lambda i: (i, 0)),
            pl.BlockSpec((BLK, 1), lambda i: (i, 0)),
            pl.BlockSpec((F, F), full),
            pl.BlockSpec((1, F), full),
            pl.BlockSpec((F, P), full),
            pl.BlockSpec((1, P), full),
            pl.BlockSpec((P, F), full),
            pl.BlockSpec((1, F), full),
        ],
        out_specs=[
            pl.BlockSpec((BLK, F), lambda i: (i, 0)),
            pl.BlockSpec((BLK, F), lambda i: (i, 0)),
        ],
        out_shape=[
            jax.ShapeDtypeStruct((N, F), jnp.float32),
            jax.ShapeDtypeStruct((N, F), jnp.float32),
        ],
    )(m1a, m1b, t_t, si, so, W1, b1r, Wp, bpr, W2b, b2r)


def _fin_body(m2a_ref, m2b_ref, r_ref, si_ref, w2a_ref, out_ref):
    m2 = (m2a_ref[...] + m2b_ref[...]) * si_ref[...]
    out_ref[...] = (
        jnp.dot(m2, w2a_ref[...], preferred_element_type=jnp.float32) + r_ref[...]
    )


def _fin_call(m2a, m2b, r, si, W2a):
    grid = (N // BLK,)
    full = lambda i: (0, 0)
    return pl.pallas_call(
        _fin_body,
        grid=grid,
        in_specs=[
            pl.BlockSpec((BLK, F), lambda i: (i, 0)),
            pl.BlockSpec((BLK, F), lambda i: (i, 0)),
            pl.BlockSpec((BLK, F), lambda i: (i, 0)),
            pl.BlockSpec((BLK, 1), lambda i: (i, 0)),
            pl.BlockSpec((F, F), full),
        ],
        out_specs=pl.BlockSpec((BLK, F), lambda i: (i, 0)),
        out_shape=jax.ShapeDtypeStruct((N, F), jnp.float32),
    )(m2a, m2b, r, si, W2a)


def kernel(feats, edge_index, W1, b1, Wp, bp, W2, b2):
    src = edge_index[0]
    dst = edge_index[1]

    dout_p, din_p = _deg_kernel(src, dst)
    xs, so, si = _prescale_call(dout_p.reshape(NW, N).T, din_p.reshape(NW, N).T, feats)

    m1_p, t_p = _pass1_kernel(xs, src, dst, so.reshape(N))
    h1s, r = _mid_call(
        m1_p[0], m1_p[1], t_p.reshape(NW, N).T, si, so,
        W1, b1.reshape(1, F), Wp, bp.reshape(1, P), W2[F:], b2.reshape(1, F),
    )

    m2_p = _pass2_kernel(h1s, src, dst)
    out = _fin_call(m2_p[0], m2_p[1], r, si, W2[:F])
    return out
